# Initial kernel scaffold; baseline (speedup 1.0000x reference)
import jax
import jax.numpy as jnp
from jax.experimental import pallas as pl


def kernel(x, edge_index, W1, b1, W2, b2):
    raise NotImplementedError("write your pallas kernel here")


# speedup vs baseline: 11.5907x; 11.5907x over previous
import functools

import jax
import jax.numpy as jnp
from jax import lax
from jax.experimental import pallas as pl
from jax.experimental.pallas import tpu as pltpu
from jax.experimental.pallas import tpu_sc as plsc

_N = 10000
_D = 128
_HD = 64
_NP = 10240
_NC, _NS = 2, 16
_K = 128
_CH = 160
_EP = _NS * _CH * _K
_CHD = _CH // 2
_RPT = _NP // _NS
_DUMMY = _NP - 1

_mesh = plsc.VectorSubcoreMesh(core_axis_name="c", subcore_axis_name="s")
_sc_params = pltpu.CompilerParams(use_tc_tiling_on_sc=False)


@functools.partial(
    pl.kernel,
    out_type=jax.ShapeDtypeStruct((_NC, _NP, 16), jnp.float32),
    mesh=_mesh,
    scratch_types=[
        pltpu.VMEM((_CHD, _K), jnp.int32),
        pltpu.VMEM((_K, 16), jnp.float32),
        pltpu.VMEM_SHARED((_NP, 16), jnp.float32),
    ],
    compiler_params=_sc_params,
)
def _deg_sc(colp, ones, z16, degp, colv, ones_v, dacc):
    c = lax.axis_index("c")
    s = lax.axis_index("s")
    pltpu.sync_copy(colp.at[s].at[pl.ds(c * _CHD, _CHD)], colv)
    pltpu.sync_copy(ones, ones_v)
    pltpu.sync_copy(z16.at[pl.ds(s * _RPT, _RPT)], dacc.at[pl.ds(s * _RPT, _RPT)])
    plsc.subcore_barrier()

    def body(j, carry):
        pltpu.sync_copy(ones_v, dacc.at[colv.at[j]], add=True)
        return carry

    lax.fori_loop(0, _CHD, body, 0)
    plsc.subcore_barrier()
    pltpu.sync_copy(dacc.at[pl.ds(s * _RPT, _RPT)],
                    degp.at[c].at[pl.ds(s * _RPT, _RPT)])


@functools.partial(
    pl.kernel,
    out_type=jax.ShapeDtypeStruct((_NC, _NP, _HD), jnp.float32),
    mesh=_mesh,
    scratch_types=[
        pltpu.VMEM((_CH, _K), jnp.int32),
        pltpu.VMEM((_CH, _K), jnp.int32),
        pltpu.VMEM((_K, _HD), jnp.float32),
        pltpu.VMEM((_K, _HD), jnp.float32),
        pltpu.VMEM_SHARED((_NP, _HD), jnp.float32),
        pltpu.SemaphoreType.DMA,
        pltpu.SemaphoreType.DMA,
        pltpu.SemaphoreType.DMA,
    ],
    compiler_params=_sc_params,
)
def _agg_sc(y, rowp, colp, z2d, parts, rowv, colv, buf0, buf1, acc, gsem, s0, s1):
    c = lax.axis_index("c")
    s = lax.axis_index("s")
    yc = y.at[c]
    pltpu.sync_copy(rowp.at[s], rowv)
    pltpu.sync_copy(colp.at[s], colv)
    pltpu.sync_copy(z2d.at[pl.ds(s * _RPT, _RPT)], acc.at[pl.ds(s * _RPT, _RPT)])
    plsc.subcore_barrier()

    npairs = _CH // 2
    pltpu.async_copy(yc.at[rowv.at[0]], buf0, gsem)

    def pair(p, carry):
        j = 2 * p
        pltpu.make_async_copy(yc.at[rowv.at[j]], buf0, gsem).wait()

        @pl.when(p > 0)
        def _wait_prev_scatter():
            pltpu.make_async_copy(buf1, acc.at[colv.at[j - 1]], s1).wait()

        pltpu.async_copy(yc.at[rowv.at[j + 1]], buf1, gsem)
        pltpu.async_copy(buf0, acc.at[colv.at[j]], s0, add=True)
        pltpu.make_async_copy(yc.at[rowv.at[j + 1]], buf1, gsem).wait()
        pltpu.make_async_copy(buf0, acc.at[colv.at[j]], s0).wait()

        @pl.when(p < npairs - 1)
        def _next_gather():
            pltpu.async_copy(yc.at[rowv.at[j + 2]], buf0, gsem)

        pltpu.async_copy(buf1, acc.at[colv.at[j + 1]], s1, add=True)
        return carry

    lax.fori_loop(0, npairs, pair, 0)
    pltpu.make_async_copy(buf1, acc.at[colv.at[_CH - 1]], s1).wait()
    plsc.subcore_barrier()
    pltpu.sync_copy(acc.at[pl.ds(s * _RPT, _RPT)],
                    parts.at[c].at[pl.ds(s * _RPT, _RPT)])


_BLK = 1024
_G = _NP // _BLK


def _prep_body(degp, xp, w1, b1, dis_o, dinv_o, y1_o, st1_o):
    deg = 1.0 + degp[0][:, 0:1] + degp[1][:, 0:1]
    dinv = 1.0 / deg
    dis = lax.rsqrt(deg)
    xw = jnp.dot(xp[...], w1[...])
    y = xw * dis
    dis_o[...] = dis
    dinv_o[...] = dinv
    y1_o[0] = y[:, :_HD]
    y1_o[1] = y[:, _HD:]
    st1_o[...] = xw * dinv + b1[...]


_prep = pl.pallas_call(
    _prep_body,
    grid=(_G,),
    in_specs=[
        pl.BlockSpec((_NC, _BLK, 16), lambda i: (0, i, 0)),
        pl.BlockSpec((_BLK, _D), lambda i: (i, 0)),
        pl.BlockSpec((_D, _D), lambda i: (0, 0)),
        pl.BlockSpec((1, _D), lambda i: (0, 0)),
    ],
    out_specs=[
        pl.BlockSpec((_BLK, 1), lambda i: (i, 0)),
        pl.BlockSpec((_BLK, 1), lambda i: (i, 0)),
        pl.BlockSpec((_NC, _BLK, _HD), lambda i: (0, i, 0)),
        pl.BlockSpec((_BLK, _D), lambda i: (i, 0)),
    ],
    out_shape=[
        jax.ShapeDtypeStruct((_NP, 1), jnp.float32),
        jax.ShapeDtypeStruct((_NP, 1), jnp.float32),
        jax.ShapeDtypeStruct((_NC, _NP, _HD), jnp.float32),
        jax.ShapeDtypeStruct((_NP, _D), jnp.float32),
    ],
)


def _mid_body(parts, dis, dinv, st1, w2, b2, y2_o, st2_o):
    agg = jnp.concatenate([parts[0], parts[1]], axis=1)
    h = jnp.maximum(dis[...] * agg + st1[...], 0.0)
    xw2 = jnp.dot(h, w2[...])
    y2 = xw2 * dis[...]
    y2_o[0] = y2[:, :_HD]
    y2_o[1] = y2[:, _HD:]
    st2_o[...] = xw2 * dinv[...] + b2[...]


_mid = pl.pallas_call(
    _mid_body,
    grid=(_G,),
    in_specs=[
        pl.BlockSpec((_NC, _BLK, _HD), lambda i: (0, i, 0)),
        pl.BlockSpec((_BLK, 1), lambda i: (i, 0)),
        pl.BlockSpec((_BLK, 1), lambda i: (i, 0)),
        pl.BlockSpec((_BLK, _D), lambda i: (i, 0)),
        pl.BlockSpec((_D, _D), lambda i: (0, 0)),
        pl.BlockSpec((1, _D), lambda i: (0, 0)),
    ],
    out_specs=[
        pl.BlockSpec((_NC, _BLK, _HD), lambda i: (0, i, 0)),
        pl.BlockSpec((_BLK, _D), lambda i: (i, 0)),
    ],
    out_shape=[
        jax.ShapeDtypeStruct((_NC, _NP, _HD), jnp.float32),
        jax.ShapeDtypeStruct((_NP, _D), jnp.float32),
    ],
)


def _final_body(parts, dis, st2, out_o):
    agg = jnp.concatenate([parts[0], parts[1]], axis=1)
    out_o[...] = dis[...] * agg + st2[...]


_final = pl.pallas_call(
    _final_body,
    grid=(_G,),
    in_specs=[
        pl.BlockSpec((_NC, _BLK, _HD), lambda i: (0, i, 0)),
        pl.BlockSpec((_BLK, 1), lambda i: (i, 0)),
        pl.BlockSpec((_BLK, _D), lambda i: (i, 0)),
    ],
    out_specs=pl.BlockSpec((_BLK, _D), lambda i: (i, 0)),
    out_shape=jax.ShapeDtypeStruct((_NP, _D), jnp.float32),
)


def kernel(x, edge_index, W1, b1, W2, b2):
    f32 = jnp.float32
    n, d = x.shape
    row = edge_index[0].astype(jnp.int32)
    col = edge_index[1].astype(jnp.int32)
    padn = _EP - row.shape[0]
    dummy = jnp.full((padn,), _DUMMY, jnp.int32)
    rowp = jnp.concatenate([row, dummy]).reshape(_NS, _CH, _K)
    colp = jnp.concatenate([col, dummy]).reshape(_NS, _CH, _K)
    xp = jnp.zeros((_NP, d), f32).at[:n].set(x.astype(f32))
    z2d = jnp.zeros((_NP, _HD), f32)
    z16 = jnp.zeros((_NP, 16), f32)
    ones = jnp.ones((_K, 16), f32)

    degp = _deg_sc(colp, ones, z16)
    dis, dinv, y1, st1 = _prep(degp, xp, W1, b1.reshape(1, -1))
    parts1 = _agg_sc(y1, rowp, colp, z2d)
    y2, st2 = _mid(parts1, dis, dinv, st1, W2, b2.reshape(1, -1))
    parts2 = _agg_sc(y2, rowp, colp, z2d)
    outp = _final(parts2, dis, st2)
    return outp[:n]


# speedup vs baseline: 12.9399x; 1.1164x over previous
import functools

import jax
import jax.numpy as jnp
from jax import lax
from jax.experimental import pallas as pl
from jax.experimental.pallas import tpu as pltpu
from jax.experimental.pallas import tpu_sc as plsc

_N = 10000
_D = 128
_HD = 64
_NP = 10240
_NC, _NS = 2, 16
_K = 128
_CH = 160
_EP = _NS * _CH * _K
_CHD = _CH // 2
_RPT = _NP // _NS
_DUMMY = _NP - 1

_mesh = plsc.VectorSubcoreMesh(core_axis_name="c", subcore_axis_name="s")
_sc_params = pltpu.CompilerParams(use_tc_tiling_on_sc=False)


@functools.partial(
    pl.kernel,
    out_type=jax.ShapeDtypeStruct((_NC, _NP, 16), jnp.float32),
    mesh=_mesh,
    scratch_types=[
        pltpu.VMEM((_CHD, _K), jnp.int32),
        pltpu.VMEM((_K, 16), jnp.float32),
        pltpu.VMEM((_K, 16), jnp.float32),
        pltpu.VMEM_SHARED((_NP, 16), jnp.float32),
    ],
    compiler_params=_sc_params,
)
def _deg_sc(colp, degp, colv, ones_v, zv, dacc):
    c = lax.axis_index("c")
    s = lax.axis_index("s")
    pltpu.sync_copy(colp.at[s].at[pl.ds(c * _CHD, _CHD)], colv)

    def fill(r, carry):
        ones_v[r, pl.ds(0, 16)] = jnp.full((16,), 1.0, jnp.float32)
        zv[r, pl.ds(0, 16)] = jnp.zeros((16,), jnp.float32)
        return carry

    lax.fori_loop(0, _K, fill, 0)
    for t in range(_RPT // _K):
        pltpu.sync_copy(zv, dacc.at[pl.ds(s * _RPT + t * _K, _K)])
    plsc.subcore_barrier()

    def body(j, carry):
        pltpu.sync_copy(ones_v, dacc.at[colv.at[j]], add=True)
        return carry

    lax.fori_loop(0, _CHD, body, 0)
    plsc.subcore_barrier()
    pltpu.sync_copy(dacc.at[pl.ds(s * _RPT, _RPT)],
                    degp.at[c].at[pl.ds(s * _RPT, _RPT)])


_NBUF = 4


@functools.partial(
    pl.kernel,
    out_type=jax.ShapeDtypeStruct((_NC, _NP, _HD), jnp.float32),
    mesh=_mesh,
    scratch_types=[
        pltpu.VMEM((_CH, _K), jnp.int32),
        pltpu.VMEM((_CH, _K), jnp.int32),
        [pltpu.VMEM((_K, _HD), jnp.float32)] * _NBUF,
        [pltpu.SemaphoreType.DMA] * _NBUF,
        [pltpu.SemaphoreType.DMA] * _NBUF,
        pltpu.VMEM_SHARED((_NP, _HD), jnp.float32),
    ],
    compiler_params=_sc_params,
)
def _agg_sc(y, rowp, colp, parts, rowv, colv, bufs, gsems, ssems, acc):
    c = lax.axis_index("c")
    s = lax.axis_index("s")
    yc = y.at[c]
    pltpu.sync_copy(rowp.at[s], rowv)
    pltpu.sync_copy(colp.at[s], colv)

    def zfill(r, carry):
        for q in range(_HD // 16):
            bufs[0][r, pl.ds(q * 16, 16)] = jnp.zeros((16,), jnp.float32)
        return carry

    lax.fori_loop(0, _K, zfill, 0)
    for t in range(_RPT // _K):
        pltpu.sync_copy(bufs[0], acc.at[pl.ds(s * _RPT + t * _K, _K)])
    plsc.subcore_barrier()

    half = _NBUF // 2

    def gather(k, b):
        pltpu.async_copy(yc.at[rowv.at[k]], bufs[b], gsems[b])

    def scatter(k, b):
        pltpu.async_copy(bufs[b], acc.at[colv.at[k]], ssems[b], add=True)

    def step(k, b, first, last):
        pltpu.make_async_copy(yc.at[rowv.at[k]], bufs[b], gsems[b]).wait()
        scatter(k, b)
        b4 = (b + half) % _NBUF
        if not first:
            pltpu.make_async_copy(bufs[b4], acc.at[colv.at[k - half]],
                                  ssems[b4]).wait()
        if not last:
            gather(k + half, b4)

    for b in range(half):
        gather(b, b)
    for k in range(_NBUF):
        step(k, k % _NBUF, first=(k < half), last=False)

    def block(j, carry):
        k0 = j * _NBUF
        for b in range(_NBUF):
            step(k0 + b, b, first=False, last=False)
        return carry

    lax.fori_loop(1, _CH // _NBUF - 1, block, 0)

    for b in range(_NBUF):
        step(_CH - _NBUF + b, b, first=False, last=(b >= half))
    for b in range(half):
        b4 = (b + half) % _NBUF
        pltpu.make_async_copy(bufs[b4], acc.at[colv.at[_CH - half + b]],
                              ssems[b4]).wait()

    plsc.subcore_barrier()
    pltpu.sync_copy(acc.at[pl.ds(s * _RPT, _RPT)],
                    parts.at[c].at[pl.ds(s * _RPT, _RPT)])


_BLK = 1024
_G = _NP // _BLK


def _prep_body(degp, xp, w1, b1, dis_o, dinv_o, y1_o, st1_o):
    deg = 1.0 + degp[0][:, 0:1] + degp[1][:, 0:1]
    dinv = 1.0 / deg
    dis = lax.rsqrt(deg)
    xw = jnp.dot(xp[...], w1[...])
    y = xw * dis
    dis_o[...] = dis
    dinv_o[...] = dinv
    y1_o[0] = y[:, :_HD]
    y1_o[1] = y[:, _HD:]
    st1_o[...] = xw * dinv + b1[...]


_prep = pl.pallas_call(
    _prep_body,
    grid=(_G,),
    in_specs=[
        pl.BlockSpec((_NC, _BLK, 16), lambda i: (0, i, 0)),
        pl.BlockSpec((_BLK, _D), lambda i: (i, 0)),
        pl.BlockSpec((_D, _D), lambda i: (0, 0)),
        pl.BlockSpec((1, _D), lambda i: (0, 0)),
    ],
    out_specs=[
        pl.BlockSpec((_BLK, 1), lambda i: (i, 0)),
        pl.BlockSpec((_BLK, 1), lambda i: (i, 0)),
        pl.BlockSpec((_NC, _BLK, _HD), lambda i: (0, i, 0)),
        pl.BlockSpec((_BLK, _D), lambda i: (i, 0)),
    ],
    out_shape=[
        jax.ShapeDtypeStruct((_NP, 1), jnp.float32),
        jax.ShapeDtypeStruct((_NP, 1), jnp.float32),
        jax.ShapeDtypeStruct((_NC, _NP, _HD), jnp.float32),
        jax.ShapeDtypeStruct((_NP, _D), jnp.float32),
    ],
)


def _mid_body(parts, dis, dinv, st1, w2, b2, y2_o, st2_o):
    agg = jnp.concatenate([parts[0], parts[1]], axis=1)
    h = jnp.maximum(dis[...] * agg + st1[...], 0.0)
    xw2 = jnp.dot(h, w2[...])
    y2 = xw2 * dis[...]
    y2_o[0] = y2[:, :_HD]
    y2_o[1] = y2[:, _HD:]
    st2_o[...] = xw2 * dinv[...] + b2[...]


_mid = pl.pallas_call(
    _mid_body,
    grid=(_G,),
    in_specs=[
        pl.BlockSpec((_NC, _BLK, _HD), lambda i: (0, i, 0)),
        pl.BlockSpec((_BLK, 1), lambda i: (i, 0)),
        pl.BlockSpec((_BLK, 1), lambda i: (i, 0)),
        pl.BlockSpec((_BLK, _D), lambda i: (i, 0)),
        pl.BlockSpec((_D, _D), lambda i: (0, 0)),
        pl.BlockSpec((1, _D), lambda i: (0, 0)),
    ],
    out_specs=[
        pl.BlockSpec((_NC, _BLK, _HD), lambda i: (0, i, 0)),
        pl.BlockSpec((_BLK, _D), lambda i: (i, 0)),
    ],
    out_shape=[
        jax.ShapeDtypeStruct((_NC, _NP, _HD), jnp.float32),
        jax.ShapeDtypeStruct((_NP, _D), jnp.float32),
    ],
)


def _final_body(parts, dis, st2, out_o):
    agg = jnp.concatenate([parts[0], parts[1]], axis=1)
    out_o[...] = dis[...] * agg + st2[...]


_final = pl.pallas_call(
    _final_body,
    grid=(_G,),
    in_specs=[
        pl.BlockSpec((_NC, _BLK, _HD), lambda i: (0, i, 0)),
        pl.BlockSpec((_BLK, 1), lambda i: (i, 0)),
        pl.BlockSpec((_BLK, _D), lambda i: (i, 0)),
    ],
    out_specs=pl.BlockSpec((_BLK, _D), lambda i: (i, 0)),
    out_shape=jax.ShapeDtypeStruct((_NP, _D), jnp.float32),
)


def kernel(x, edge_index, W1, b1, W2, b2):
    f32 = jnp.float32
    n, d = x.shape
    row = edge_index[0].astype(jnp.int32)
    col = edge_index[1].astype(jnp.int32)
    padn = _EP - row.shape[0]
    dummy = jnp.full((padn,), _DUMMY, jnp.int32)
    rowp = jnp.concatenate([row, dummy]).reshape(_NS, _CH, _K)
    colp = jnp.concatenate([col, dummy]).reshape(_NS, _CH, _K)
    xp = jnp.zeros((_NP, d), f32).at[:n].set(x.astype(f32))
    degp = _deg_sc(colp)
    dis, dinv, y1, st1 = _prep(degp, xp, W1, b1.reshape(1, -1))
    parts1 = _agg_sc(y1, rowp, colp)
    y2, st2 = _mid(parts1, dis, dinv, st1, W2, b2.reshape(1, -1))
    parts2 = _agg_sc(y2, rowp, colp)
    outp = _final(parts2, dis, st2)
    return outp[:n]


# speedup vs baseline: 13.9078x; 1.0748x over previous
import functools

import jax
import jax.numpy as jnp
from jax import lax
from jax.experimental import pallas as pl
from jax.experimental.pallas import tpu as pltpu
from jax.experimental.pallas import tpu_sc as plsc

_N = 10000
_D = 128
_HD = 64
_NP = 10240
_NC, _NS = 2, 16
_K = 128
_CH = 160
_EP = _NS * _CH * _K
_CHD = _CH // 2
_RPT = _NP // _NS
_DUMMY = _NP - 1
_NBUF = 4

_mesh = plsc.VectorSubcoreMesh(core_axis_name="c", subcore_axis_name="s")
_sc_params = pltpu.CompilerParams(use_tc_tiling_on_sc=False)


@functools.partial(
    pl.kernel,
    out_type=jax.ShapeDtypeStruct((_NC, _NP, 16), jnp.float32),
    mesh=_mesh,
    scratch_types=[
        pltpu.VMEM((_CHD, _K), jnp.int32),
        pltpu.VMEM((_K, 16), jnp.float32),
        pltpu.VMEM((_K, 16), jnp.float32),
        pltpu.VMEM_SHARED((_NP, 16), jnp.float32),
    ],
    compiler_params=_sc_params,
)
def _deg_sc(colp, degp, colv, ones_v, zv, dacc):
    c = lax.axis_index("c")
    s = lax.axis_index("s")
    pltpu.sync_copy(colp.at[s].at[pl.ds(c * _CHD, _CHD)], colv)

    def fill(r, carry):
        ones_v[r, pl.ds(0, 16)] = jnp.full((16,), 1.0, jnp.float32)
        zv[r, pl.ds(0, 16)] = jnp.zeros((16,), jnp.float32)
        return carry

    lax.fori_loop(0, _K, fill, 0)
    for t in range(_RPT // _K):
        pltpu.sync_copy(zv, dacc.at[pl.ds(s * _RPT + t * _K, _K)])
    plsc.subcore_barrier()

    def body(j, carry):
        pltpu.sync_copy(ones_v, dacc.at[colv.at[j]], add=True)
        return carry

    lax.fori_loop(0, _CHD, body, 0)
    plsc.subcore_barrier()
    pltpu.sync_copy(dacc.at[pl.ds(s * _RPT, _RPT)],
                    degp.at[c].at[pl.ds(s * _RPT, _RPT)])


@functools.partial(
    pl.kernel,
    out_type=jax.ShapeDtypeStruct((_NC, _NP, _HD), jnp.float32),
    mesh=_mesh,
    scratch_types=[
        pltpu.VMEM((_CH, _K), jnp.int32),
        pltpu.VMEM((_CH, _K), jnp.int32),
        [pltpu.VMEM((_K, _HD), jnp.float32)] * _NBUF,
        [pltpu.SemaphoreType.DMA] * _NBUF,
        [pltpu.SemaphoreType.DMA] * _NBUF,
        pltpu.VMEM_SHARED((_NP, _HD), jnp.float32),
    ],
    compiler_params=_sc_params,
)
def _agg_sc(z, rowp, colp, parts, rowv, colv, bufs, gsems, ssems, acc):
    c = lax.axis_index("c")
    s = lax.axis_index("s")
    zsp = z.at[c]
    pltpu.sync_copy(rowp.at[s], rowv)
    pltpu.sync_copy(colp.at[s], colv)

    def zfill(r, carry):
        for q in range(_HD // 16):
            bufs[0][r, pl.ds(q * 16, 16)] = jnp.zeros((16,), jnp.float32)
        return carry

    lax.fori_loop(0, _K, zfill, 0)
    for t in range(_RPT // _K):
        pltpu.sync_copy(bufs[0], acc.at[pl.ds(s * _RPT + t * _K, _K)])
    plsc.subcore_barrier()

    half = _NBUF // 2

    def gather(k, b):
        pltpu.async_copy(zsp.at[rowv.at[k]], bufs[b], gsems[b])

    def scatter(k, b):
        pltpu.async_copy(bufs[b], acc.at[colv.at[k]], ssems[b], add=True)

    def step(k, b, first, last):
        pltpu.make_async_copy(zsp.at[rowv.at[k]], bufs[b], gsems[b]).wait()
        scatter(k, b)
        b4 = (b + half) % _NBUF
        if not first:
            pltpu.make_async_copy(bufs[b4], acc.at[colv.at[k - half]],
                                  ssems[b4]).wait()
        if not last:
            gather(k + half, b4)

    for b in range(half):
        gather(b, b)
    for k in range(_NBUF):
        step(k, k % _NBUF, first=(k < half), last=False)

    def block(j, carry):
        k0 = j * _NBUF
        for b in range(_NBUF):
            step(k0 + b, b, first=False, last=False)
        return carry

    lax.fori_loop(1, _CH // _NBUF - 1, block, 0)

    for b in range(_NBUF):
        step(_CH - _NBUF + b, b, first=False, last=(b >= half))
    for b in range(half):
        b4 = (b + half) % _NBUF
        pltpu.make_async_copy(bufs[b4], acc.at[colv.at[_CH - half + b]],
                              ssems[b4]).wait()

    plsc.subcore_barrier()
    pltpu.sync_copy(acc.at[pl.ds(s * _RPT, _RPT)],
                    parts.at[c].at[pl.ds(s * _RPT, _RPT)])


_BLK = 1024
_G = _NP // _BLK


def _prep_body(degp, xp, dis_o, dinv_o, z0_o):
    deg = 1.0 + degp[0][:, 0:1] + degp[1][:, 0:1]
    dinv = 1.0 / deg
    dis = lax.rsqrt(deg)
    z0 = xp[...] * dis
    dis_o[...] = dis
    dinv_o[...] = dinv
    z0_o[0] = z0[:, :_HD]
    z0_o[1] = z0[:, _HD:]


_prep = pl.pallas_call(
    _prep_body,
    grid=(_G,),
    in_specs=[
        pl.BlockSpec((_NC, _BLK, 16), lambda i: (0, i, 0)),
        pl.BlockSpec((_BLK, _D), lambda i: (i, 0)),
    ],
    out_specs=[
        pl.BlockSpec((_BLK, 1), lambda i: (i, 0)),
        pl.BlockSpec((_BLK, 1), lambda i: (i, 0)),
        pl.BlockSpec((_NC, _BLK, _HD), lambda i: (0, i, 0)),
    ],
    out_shape=[
        jax.ShapeDtypeStruct((_NP, 1), jnp.float32),
        jax.ShapeDtypeStruct((_NP, 1), jnp.float32),
        jax.ShapeDtypeStruct((_NC, _NP, _HD), jnp.float32),
    ],
)


def _layer_body(parts, base, dis, dinv, w, b, flag, o_o, z_o, h_o):
    g = jnp.concatenate([parts[0], parts[1]], axis=1)
    t1 = jnp.dot(g, w[...])
    t2 = jnp.dot(base[...], w[...])
    o = dis[...] * t1 + dinv[...] * t2 + b[...]
    h = jnp.where(flag[...] > 0.0, jnp.maximum(o, 0.0), o)
    z = h * dis[...]
    o_o[...] = o
    h_o[...] = h
    z_o[0] = z[:, :_HD]
    z_o[1] = z[:, _HD:]


_layer = pl.pallas_call(
    _layer_body,
    grid=(_G,),
    in_specs=[
        pl.BlockSpec((_NC, _BLK, _HD), lambda i: (0, i, 0)),
        pl.BlockSpec((_BLK, _D), lambda i: (i, 0)),
        pl.BlockSpec((_BLK, 1), lambda i: (i, 0)),
        pl.BlockSpec((_BLK, 1), lambda i: (i, 0)),
        pl.BlockSpec((_D, _D), lambda i: (0, 0)),
        pl.BlockSpec((1, _D), lambda i: (0, 0)),
        pl.BlockSpec((1, 1), lambda i: (0, 0)),
    ],
    out_specs=[
        pl.BlockSpec((_BLK, _D), lambda i: (i, 0)),
        pl.BlockSpec((_NC, _BLK, _HD), lambda i: (0, i, 0)),
        pl.BlockSpec((_BLK, _D), lambda i: (i, 0)),
    ],
    out_shape=[
        jax.ShapeDtypeStruct((_NP, _D), jnp.float32),
        jax.ShapeDtypeStruct((_NC, _NP, _HD), jnp.float32),
        jax.ShapeDtypeStruct((_NP, _D), jnp.float32),
    ],
)


def kernel(x, edge_index, W1, b1, W2, b2):
    f32 = jnp.float32
    n, d = x.shape
    row = edge_index[0].astype(jnp.int32)
    col = edge_index[1].astype(jnp.int32)
    padn = _EP - row.shape[0]
    dummy = jnp.full((padn,), _DUMMY, jnp.int32)
    rowp = jnp.concatenate([row, dummy]).reshape(_NS, _CH, _K)
    colp = jnp.concatenate([col, dummy]).reshape(_NS, _CH, _K)
    xp = jnp.zeros((_NP, d), f32).at[:n].set(x.astype(f32))

    degp = _deg_sc(colp)
    dis, dinv, z0 = _prep(degp, xp)

    wstack = jnp.stack([W1, W2])
    bstack = jnp.stack([b1.reshape(1, -1), b2.reshape(1, -1)])
    fstack = jnp.array([[[1.0]], [[0.0]]], f32)

    def body(carry, xs):
        z, base, _ = carry
        w, bvec, flag = xs
        parts = _agg_sc(z, rowp, colp)
        o, znext, hnext = _layer(parts, base, dis, dinv, w, bvec, flag)
        return (znext, hnext, o), None

    init = (z0, xp, jnp.zeros((_NP, _D), f32))
    (zf, hf, o), _ = lax.scan(body, init, (wstack, bstack, fstack))
    return o[:n]


# speedup vs baseline: 15.1985x; 1.0928x over previous
import functools

import jax
import jax.numpy as jnp
from jax import lax
from jax.experimental import pallas as pl
from jax.experimental.pallas import tpu as pltpu
from jax.experimental.pallas import tpu_sc as plsc

_N = 10000
_D = 128
_HD = 64
_NP = 10240
_NC, _NS = 2, 16
_K = 128
_CH = 160
_EP = _NS * _CH * _K
_CHD = _CH // 2
_RPT = _NP // _NS
_DUMMY = _NP - 1
_NBUF = 4

_mesh = plsc.VectorSubcoreMesh(core_axis_name="c", subcore_axis_name="s")
_sc_params = pltpu.CompilerParams(use_tc_tiling_on_sc=False, needs_layout_passes=False)


@functools.partial(
    pl.kernel,
    out_type=jax.ShapeDtypeStruct((_NC, _NP, 16), jnp.float32),
    mesh=_mesh,
    scratch_types=[
        pltpu.VMEM((_CHD, _K), jnp.int32),
        pltpu.VMEM((_K, 16), jnp.float32),
        pltpu.VMEM((_K, 16), jnp.float32),
        pltpu.VMEM_SHARED((_NP, 16), jnp.float32),
    ],
    compiler_params=_sc_params,
)
def _deg_sc(colp, degp, colv, ones_v, zv, dacc):
    c = lax.axis_index("c")
    s = lax.axis_index("s")
    pltpu.sync_copy(colp.at[s].at[pl.ds(c * _CHD, _CHD)], colv)

    def fill(r, carry):
        ones_v[r, pl.ds(0, 16)] = jnp.full((16,), 1.0, jnp.float32)
        zv[r, pl.ds(0, 16)] = jnp.zeros((16,), jnp.float32)
        return carry

    lax.fori_loop(0, _K, fill, 0)
    for t in range(_RPT // _K):
        pltpu.sync_copy(zv, dacc.at[pl.ds(s * _RPT + t * _K, _K)])
    plsc.subcore_barrier()

    def body(j, carry):
        pltpu.sync_copy(ones_v, dacc.at[colv.at[j]], add=True)
        return carry

    lax.fori_loop(0, _CHD, body, 0)
    plsc.subcore_barrier()
    pltpu.sync_copy(dacc.at[pl.ds(s * _RPT, _RPT)],
                    degp.at[c].at[pl.ds(s * _RPT, _RPT)])


@functools.partial(
    pl.kernel,
    out_type=jax.ShapeDtypeStruct((_NC, _NP, _HD), jnp.float32),
    mesh=_mesh,
    scratch_types=[
        pltpu.VMEM((_CH, _K), jnp.int32),
        pltpu.VMEM((_CH, _K), jnp.int32),
        [pltpu.VMEM((_K, _HD), jnp.bfloat16)] * 2,
        [pltpu.VMEM((_K, _HD), jnp.float32)] * 2,
        [pltpu.SemaphoreType.DMA] * 2,
        [pltpu.SemaphoreType.DMA] * 2,
        pltpu.VMEM_SHARED((_NP, _HD), jnp.float32),
    ],
    compiler_params=_sc_params,
)
def _agg_sc(z, rowp, colp, parts, rowv, colv, gbufs, fbufs, gsems, ssems, acc):
    c = lax.axis_index("c")
    s = lax.axis_index("s")
    zsp = z.at[c]
    pltpu.sync_copy(rowp.at[s], rowv)
    pltpu.sync_copy(colp.at[s], colv)

    def zfill(r, carry):
        for q in range(_HD // 16):
            fbufs[0][r, pl.ds(q * 16, 16)] = jnp.zeros((16,), jnp.float32)
        return carry

    lax.fori_loop(0, _K, zfill, 0)
    for t in range(_RPT // _K):
        pltpu.sync_copy(fbufs[0], acc.at[pl.ds(s * _RPT + t * _K, _K)])
    plsc.subcore_barrier()

    def gather(k, b):
        pltpu.async_copy(zsp.at[rowv.at[k]], gbufs[b], gsems[b])

    def scatter(k, b):
        pltpu.async_copy(fbufs[b], acc.at[colv.at[k]], ssems[b], add=True)

    def convert(b):
        def crow(r, carry):
            for q in range(_HD // 32):
                v = gbufs[b][r, pl.ds(32 * q, 32)]
                lo, hi = plsc.unpack(v, format=plsc.PackFormat.INTERLEAVED)
                fbufs[b][r, pl.ds(32 * q, 16)] = lo
                fbufs[b][r, pl.ds(32 * q + 16, 16)] = hi
            return carry

        lax.fori_loop(0, _K, crow, 0, unroll=2)

    def step(k, b, first, last):
        pltpu.make_async_copy(zsp.at[rowv.at[k]], gbufs[b], gsems[b]).wait()
        if not first:
            pltpu.make_async_copy(fbufs[b], acc.at[colv.at[k - 2]],
                                  ssems[b]).wait()
        convert(b)
        if not last:
            gather(k + 2, b)
        scatter(k, b)

    gather(0, 0)
    gather(1, 1)
    step(0, 0, first=True, last=False)
    step(1, 1, first=True, last=False)

    def block(j, carry):
        k0 = 2 * j
        step(k0, 0, first=False, last=False)
        step(k0 + 1, 1, first=False, last=False)
        return carry

    lax.fori_loop(1, _CH // 2 - 1, block, 0)

    step(_CH - 2, 0, first=False, last=True)
    step(_CH - 1, 1, first=False, last=True)
    for b in range(2):
        pltpu.make_async_copy(fbufs[b], acc.at[colv.at[_CH - 2 + b]],
                              ssems[b]).wait()

    plsc.subcore_barrier()
    pltpu.sync_copy(acc.at[pl.ds(s * _RPT, _RPT)],
                    parts.at[c].at[pl.ds(s * _RPT, _RPT)])


_BLK = 1024
_G = _NP // _BLK


def _to_bf(z64):
    return z64.astype(jnp.bfloat16)


def _prep_body(degp, xp, dis_o, dinv_o, z0_o):
    deg = 1.0 + degp[0][:, 0:1] + degp[1][:, 0:1]
    dinv = 1.0 / deg
    dis = lax.rsqrt(deg)
    z0 = xp[...] * dis
    dis_o[...] = dis
    dinv_o[...] = dinv
    z0_o[0] = _to_bf(z0[:, :_HD])
    z0_o[1] = _to_bf(z0[:, _HD:])


_prep = pl.pallas_call(
    _prep_body,
    grid=(_G,),
    in_specs=[
        pl.BlockSpec((_NC, _BLK, 16), lambda i: (0, i, 0)),
        pl.BlockSpec((_BLK, _D), lambda i: (i, 0)),
    ],
    out_specs=[
        pl.BlockSpec((_BLK, 1), lambda i: (i, 0)),
        pl.BlockSpec((_BLK, 1), lambda i: (i, 0)),
        pl.BlockSpec((_NC, _BLK, _HD), lambda i: (0, i, 0)),
    ],
    out_shape=[
        jax.ShapeDtypeStruct((_NP, 1), jnp.float32),
        jax.ShapeDtypeStruct((_NP, 1), jnp.float32),
        jax.ShapeDtypeStruct((_NC, _NP, _HD), jnp.bfloat16),
    ],
)


def _layer_body(parts, base, dis, dinv, w, wp, b, flag, o_o, z_o, h_o):
    g = jnp.concatenate([parts[0], parts[1]], axis=1)
    t1 = jnp.dot(g, wp[...])
    t2 = jnp.dot(base[...], w[...])
    o = dis[...] * t1 + dinv[...] * t2 + b[...]
    h = jnp.where(flag[...] > 0.0, jnp.maximum(o, 0.0), o)
    z = h * dis[...]
    o_o[...] = o
    h_o[...] = h
    z_o[0] = _to_bf(z[:, :_HD])
    z_o[1] = _to_bf(z[:, _HD:])


_layer = pl.pallas_call(
    _layer_body,
    grid=(_G,),
    in_specs=[
        pl.BlockSpec((_NC, _BLK, _HD), lambda i: (0, i, 0)),
        pl.BlockSpec((_BLK, _D), lambda i: (i, 0)),
        pl.BlockSpec((_BLK, 1), lambda i: (i, 0)),
        pl.BlockSpec((_BLK, 1), lambda i: (i, 0)),
        pl.BlockSpec((_D, _D), lambda i: (0, 0)),
        pl.BlockSpec((_D, _D), lambda i: (0, 0)),
        pl.BlockSpec((1, _D), lambda i: (0, 0)),
        pl.BlockSpec((1, 1), lambda i: (0, 0)),
    ],
    out_specs=[
        pl.BlockSpec((_BLK, _D), lambda i: (i, 0)),
        pl.BlockSpec((_NC, _BLK, _HD), lambda i: (0, i, 0)),
        pl.BlockSpec((_BLK, _D), lambda i: (i, 0)),
    ],
    out_shape=[
        jax.ShapeDtypeStruct((_NP, _D), jnp.float32),
        jax.ShapeDtypeStruct((_NC, _NP, _HD), jnp.bfloat16),
        jax.ShapeDtypeStruct((_NP, _D), jnp.float32),
    ],
)


def kernel(x, edge_index, W1, b1, W2, b2):
    f32 = jnp.float32
    n, d = x.shape
    row = edge_index[0].astype(jnp.int32)
    col = edge_index[1].astype(jnp.int32)
    padn = _EP - row.shape[0]
    dummy = jnp.full((padn,), _DUMMY, jnp.int32)
    rowp = jnp.concatenate([row, dummy]).reshape(_NS, _CH, _K)
    colp = jnp.concatenate([col, dummy]).reshape(_NS, _CH, _K)
    xp = jnp.zeros((_NP, d), f32).at[:n].set(x.astype(f32))

    degp = _deg_sc(colp)
    dis, dinv, z0 = _prep(degp, xp)

    wstack = jnp.stack([W1, W2])
    perm = []
    for h in range(_D // 32):
        base32 = 32 * h
        perm += [base32 + 2 * m for m in range(16)]
        perm += [base32 + 2 * m + 1 for m in range(16)]
    pvec = jnp.array(perm, jnp.int32)
    wpstack = wstack[:, pvec, :]
    bstack = jnp.stack([b1.reshape(1, -1), b2.reshape(1, -1)])
    fstack = jnp.array([[[1.0]], [[0.0]]], f32)

    def body(carry, xs):
        z, base, _ = carry
        w, wp, bvec, flag = xs
        parts = _agg_sc(z, rowp, colp)
        o, znext, hnext = _layer(parts, base, dis, dinv, w, wp, bvec, flag)
        return (znext, hnext, o), None

    init = (z0, xp, jnp.zeros((_NP, _D), f32))
    (zf, hf, o), _ = lax.scan(body, init, (wstack, wpstack, bstack, fstack))
    return o[:n]


# speedup vs baseline: 15.6270x; 1.0282x over previous
import functools

import jax
import jax.numpy as jnp
from jax import lax
from jax.experimental import pallas as pl
from jax.experimental.pallas import tpu as pltpu
from jax.experimental.pallas import tpu_sc as plsc

_N = 10000
_D = 128
_HD = 64
_NP = 10240
_NC, _NS = 2, 16
_K = 128
_CH = 160
_EP = _NS * _CH * _K
_CHD = _CH // 2
_RPT = _NP // _NS
_DUMMY = _NP - 1
_NBUF = 4

_mesh = plsc.VectorSubcoreMesh(core_axis_name="c", subcore_axis_name="s")
_sc_params = pltpu.CompilerParams(use_tc_tiling_on_sc=False, needs_layout_passes=False)


@functools.partial(
    pl.kernel,
    out_type=jax.ShapeDtypeStruct((_NC, _NP, 16), jnp.float32),
    mesh=_mesh,
    scratch_types=[
        pltpu.VMEM((_CHD, _K), jnp.int32),
        pltpu.VMEM((_K, 16), jnp.float32),
        pltpu.VMEM((_K, 16), jnp.float32),
        pltpu.VMEM_SHARED((_NP, 16), jnp.float32),
    ],
    compiler_params=_sc_params,
)
def _deg_sc(colp, degp, colv, ones_v, zv, dacc):
    c = lax.axis_index("c")
    s = lax.axis_index("s")
    pltpu.sync_copy(colp.at[s].at[pl.ds(c * _CHD, _CHD)], colv)

    def fill(r, carry):
        ones_v[r, pl.ds(0, 16)] = jnp.full((16,), 1.0, jnp.float32)
        zv[r, pl.ds(0, 16)] = jnp.zeros((16,), jnp.float32)
        return carry

    lax.fori_loop(0, _K, fill, 0)
    for t in range(_RPT // _K):
        pltpu.sync_copy(zv, dacc.at[pl.ds(s * _RPT + t * _K, _K)])
    plsc.subcore_barrier()

    def body(j, carry):
        pltpu.sync_copy(ones_v, dacc.at[colv.at[j]], add=True)
        return carry

    lax.fori_loop(0, _CHD, body, 0)
    plsc.subcore_barrier()
    pltpu.sync_copy(dacc.at[pl.ds(s * _RPT, _RPT)],
                    degp.at[c].at[pl.ds(s * _RPT, _RPT)])


@functools.partial(
    pl.kernel,
    out_type=jax.ShapeDtypeStruct((_NC, _NP, _HD), jnp.float32),
    mesh=_mesh,
    scratch_types=[
        pltpu.VMEM((_CH, _K), jnp.int32),
        pltpu.VMEM((_CH, _K), jnp.int32),
        [pltpu.VMEM((_K, _HD), jnp.bfloat16)] * 4,
        [pltpu.VMEM((_K, _HD), jnp.float32)] * 2,
        [pltpu.SemaphoreType.DMA] * 4,
        [pltpu.SemaphoreType.DMA] * 2,
        pltpu.VMEM_SHARED((_NP, _HD), jnp.float32),
    ],
    compiler_params=_sc_params,
)
def _agg_sc(z, rowp, colp, parts, rowv, colv, gbufs, fbufs, gsems, ssems, acc):
    c = lax.axis_index("c")
    s = lax.axis_index("s")
    zsp = z.at[c]
    pltpu.sync_copy(rowp.at[s], rowv)
    pltpu.sync_copy(colp.at[s], colv)

    def zfill(r, carry):
        for q in range(_HD // 16):
            fbufs[0][r, pl.ds(q * 16, 16)] = jnp.zeros((16,), jnp.float32)
        return carry

    lax.fori_loop(0, _K, zfill, 0)
    for t in range(_RPT // _K):
        pltpu.sync_copy(fbufs[0], acc.at[pl.ds(s * _RPT + t * _K, _K)])
    plsc.subcore_barrier()

    def gather(k, b):
        pltpu.async_copy(zsp.at[rowv.at[k]], gbufs[b], gsems[b])

    def scatter(k, b):
        pltpu.async_copy(fbufs[b], acc.at[colv.at[k]], ssems[b], add=True)

    def convert(bg, bf):
        def crow(r, carry):
            for q in range(_HD // 32):
                v = gbufs[bg][r, pl.ds(32 * q, 32)]
                lo, hi = plsc.unpack(v, format=plsc.PackFormat.INTERLEAVED)
                fbufs[bf][r, pl.ds(32 * q, 16)] = lo
                fbufs[bf][r, pl.ds(32 * q + 16, 16)] = hi
            return carry

        lax.fori_loop(0, _K, crow, 0, unroll=2)

    def step(k, bg, bf, first, last):
        pltpu.make_async_copy(zsp.at[rowv.at[k]], gbufs[bg], gsems[bg]).wait()
        if not first:
            pltpu.make_async_copy(fbufs[bf], acc.at[colv.at[k - 2]],
                                  ssems[bf]).wait()
        convert(bg, bf)
        if not last:
            gather(k + 4, bg)
        scatter(k, bf)

    for b in range(4):
        gather(b, b)
    for k in range(4):
        step(k, k % 4, k % 2, first=(k < 2), last=False)

    def block(j, carry):
        k0 = 4 * j
        for b in range(4):
            step(k0 + b, b, b % 2, first=False, last=False)
        return carry

    lax.fori_loop(1, _CH // 4 - 1, block, 0)

    for b in range(4):
        step(_CH - 4 + b, b, b % 2, first=False, last=True)
    for b in range(2):
        pltpu.make_async_copy(fbufs[b], acc.at[colv.at[_CH - 2 + b]],
                              ssems[b]).wait()

    plsc.subcore_barrier()
    pltpu.sync_copy(acc.at[pl.ds(s * _RPT, _RPT)],
                    parts.at[c].at[pl.ds(s * _RPT, _RPT)])


_BLK = 1024
_G = _NP // _BLK


def _to_bf(z64):
    return z64.astype(jnp.bfloat16)


def _prep_body(degp, xp, dis_o, dinv_o, z0_o):
    deg = 1.0 + degp[0][:, 0:1] + degp[1][:, 0:1]
    dinv = 1.0 / deg
    dis = lax.rsqrt(deg)
    z0 = xp[...] * dis
    dis_o[...] = dis
    dinv_o[...] = dinv
    z0_o[0] = _to_bf(z0[:, :_HD])
    z0_o[1] = _to_bf(z0[:, _HD:])


_prep = pl.pallas_call(
    _prep_body,
    grid=(_G,),
    in_specs=[
        pl.BlockSpec((_NC, _BLK, 16), lambda i: (0, i, 0)),
        pl.BlockSpec((_BLK, _D), lambda i: (i, 0)),
    ],
    out_specs=[
        pl.BlockSpec((_BLK, 1), lambda i: (i, 0)),
        pl.BlockSpec((_BLK, 1), lambda i: (i, 0)),
        pl.BlockSpec((_NC, _BLK, _HD), lambda i: (0, i, 0)),
    ],
    out_shape=[
        jax.ShapeDtypeStruct((_NP, 1), jnp.float32),
        jax.ShapeDtypeStruct((_NP, 1), jnp.float32),
        jax.ShapeDtypeStruct((_NC, _NP, _HD), jnp.bfloat16),
    ],
)


def _layer_body(parts, base, dis, dinv, w, wp, b, flag, o_o, z_o, h_o):
    g = jnp.concatenate([parts[0], parts[1]], axis=1)
    t1 = jnp.dot(g, wp[...])
    t2 = jnp.dot(base[...], w[...])
    o = dis[...] * t1 + dinv[...] * t2 + b[...]
    h = jnp.where(flag[...] > 0.0, jnp.maximum(o, 0.0), o)
    z = h * dis[...]
    o_o[...] = o
    h_o[...] = h
    z_o[0] = _to_bf(z[:, :_HD])
    z_o[1] = _to_bf(z[:, _HD:])


_layer = pl.pallas_call(
    _layer_body,
    grid=(_G,),
    in_specs=[
        pl.BlockSpec((_NC, _BLK, _HD), lambda i: (0, i, 0)),
        pl.BlockSpec((_BLK, _D), lambda i: (i, 0)),
        pl.BlockSpec((_BLK, 1), lambda i: (i, 0)),
        pl.BlockSpec((_BLK, 1), lambda i: (i, 0)),
        pl.BlockSpec((_D, _D), lambda i: (0, 0)),
        pl.BlockSpec((_D, _D), lambda i: (0, 0)),
        pl.BlockSpec((1, _D), lambda i: (0, 0)),
        pl.BlockSpec((1, 1), lambda i: (0, 0)),
    ],
    out_specs=[
        pl.BlockSpec((_BLK, _D), lambda i: (i, 0)),
        pl.BlockSpec((_NC, _BLK, _HD), lambda i: (0, i, 0)),
        pl.BlockSpec((_BLK, _D), lambda i: (i, 0)),
    ],
    out_shape=[
        jax.ShapeDtypeStruct((_NP, _D), jnp.float32),
        jax.ShapeDtypeStruct((_NC, _NP, _HD), jnp.bfloat16),
        jax.ShapeDtypeStruct((_NP, _D), jnp.float32),
    ],
)


def kernel(x, edge_index, W1, b1, W2, b2):
    f32 = jnp.float32
    n, d = x.shape
    row = edge_index[0].astype(jnp.int32)
    col = edge_index[1].astype(jnp.int32)
    padn = _EP - row.shape[0]
    dummy = jnp.full((padn,), _DUMMY, jnp.int32)
    rowp = jnp.concatenate([row, dummy]).reshape(_NS, _CH, _K)
    colp = jnp.concatenate([col, dummy]).reshape(_NS, _CH, _K)
    xp = jnp.zeros((_NP, d), f32).at[:n].set(x.astype(f32))

    degp = _deg_sc(colp)
    dis, dinv, z0 = _prep(degp, xp)

    wstack = jnp.stack([W1, W2])
    perm = []
    for h in range(_D // 32):
        base32 = 32 * h
        perm += [base32 + 2 * m for m in range(16)]
        perm += [base32 + 2 * m + 1 for m in range(16)]
    pvec = jnp.array(perm, jnp.int32)
    wpstack = wstack[:, pvec, :]
    bstack = jnp.stack([b1.reshape(1, -1), b2.reshape(1, -1)])
    fstack = jnp.array([[[1.0]], [[0.0]]], f32)

    def body(carry, xs):
        z, base, _ = carry
        w, wp, bvec, flag = xs
        parts = _agg_sc(z, rowp, colp)
        o, znext, hnext = _layer(parts, base, dis, dinv, w, wp, bvec, flag)
        return (znext, hnext, o), None

    init = (z0, xp, jnp.zeros((_NP, _D), f32))
    (zf, hf, o), _ = lax.scan(body, init, (wstack, wpstack, bstack, fstack))
    return o[:n]
